# Initial kernel scaffold; baseline (speedup 1.0000x reference)
import jax
import jax.numpy as jnp
from jax.experimental import pallas as pl


def kernel(x, W_comp, b_comp, W_db, b_db, W_router, b_router, W1, b1, W2, b2, W_core, b_core, W_h1, b_h1, W_h2, b_h2):
    raise NotImplementedError("write your pallas kernel here")


# speedup vs baseline: 1.4870x; 1.4870x over previous
import jax
import jax.numpy as jnp
from jax.experimental import pallas as pl
from jax.experimental.pallas import tpu as pltpu

D = 1024
DFF = 2048
E = 8
B = 1024
LOOPS = 8


def _stage_a(x_ref, wc_ref, bc_ref, wd_ref, bd_ref, wr_ref, br_ref,
             ctx_ref, gates_ref):
    x = x_ref[...]
    comp = jnp.dot(x, wc_ref[...], preferred_element_type=jnp.float32) + bc_ref[...]
    ctx = jnp.dot(comp, wd_ref[...], preferred_element_type=jnp.float32) + bd_ref[...]
    ctx_ref[...] = ctx
    logits = jnp.dot(ctx, wr_ref[...], preferred_element_type=jnp.float32) + br_ref[...]
    m = jnp.max(logits, axis=-1, keepdims=True)
    ex = jnp.exp(logits - m)
    probs = ex / jnp.sum(ex, axis=-1, keepdims=True)
    lane = jax.lax.broadcasted_iota(jnp.int32, probs.shape, 1)
    v1 = jnp.max(probs, axis=-1, keepdims=True)
    i1 = jnp.argmax(probs, axis=-1)[:, None]
    masked = jnp.where(lane == i1, -jnp.inf, probs)
    v2 = jnp.max(masked, axis=-1, keepdims=True)
    i2 = jnp.argmax(masked, axis=-1)[:, None]
    s = v1 + v2
    w1 = v1 / s
    w2 = v2 / s
    gates = jnp.where(lane == i1, w1, 0.0) + jnp.where(lane == i2, w2, 0.0)
    gates_ref[...] = gates


def _stage_moe(ctx_ref, w1_ref, b1_ref, w2_ref, b2_ref, g_ref, out_ref):
    e = pl.program_id(0)
    ctx = ctx_ref[...]
    h = jnp.dot(ctx, w1_ref[0], preferred_element_type=jnp.float32) + b1_ref[0]
    h = jnp.maximum(h, 0.0)
    y = jnp.dot(h, w2_ref[0], preferred_element_type=jnp.float32) + b2_ref[0]
    gates = g_ref[...]
    lane = jax.lax.broadcasted_iota(jnp.int32, gates.shape, 1)
    g = jnp.sum(jnp.where(lane == e, gates, 0.0), axis=1, keepdims=True)
    contrib = g * y

    @pl.when(e == 0)
    def _():
        out_ref[...] = ctx + contrib

    @pl.when(e != 0)
    def _():
        out_ref[...] = out_ref[...] + contrib


def _stage_core(h_ref, wc_ref, bc_ref, wh1_ref, bh1_ref, wh2_ref, bh2_ref,
                out_ref):
    wc = wc_ref[...]
    bc = bc_ref[...]

    def body(_, h):
        dh = jnp.tanh(jnp.dot(h, wc, preferred_element_type=jnp.float32) + bc) - h
        return h + 0.1 * dh

    h = jax.lax.fori_loop(0, LOOPS, body, h_ref[...])
    hidden = jnp.dot(h, wh1_ref[...], preferred_element_type=jnp.float32) + bh1_ref[...]
    hidden = jnp.maximum(hidden, 0.0)
    out_ref[...] = jnp.dot(hidden, wh2_ref[...], preferred_element_type=jnp.float32) + bh2_ref[...]


def kernel(x, W_comp, b_comp, W_db, b_db, W_router, b_router, W1, b1, W2, b2,
           W_core, b_core, W_h1, b_h1, W_h2, b_h2):
    T = x.shape[0] * x.shape[1]
    xt = x.reshape(T, D)

    ctx, gates = pl.pallas_call(
        _stage_a,
        out_shape=(
            jax.ShapeDtypeStruct((T, D), jnp.float32),
            jax.ShapeDtypeStruct((T, E), jnp.float32),
        ),
    )(xt, W_comp, b_comp.reshape(1, D), W_db, b_db.reshape(1, D),
      W_router, b_router.reshape(1, E))

    moe = pl.pallas_call(
        _stage_moe,
        grid=(E,),
        in_specs=[
            pl.BlockSpec((T, D), lambda e: (0, 0)),
            pl.BlockSpec((1, D, DFF), lambda e: (e, 0, 0)),
            pl.BlockSpec((1, 1, DFF), lambda e: (e, 0, 0)),
            pl.BlockSpec((1, DFF, D), lambda e: (e, 0, 0)),
            pl.BlockSpec((1, 1, D), lambda e: (e, 0, 0)),
            pl.BlockSpec((T, E), lambda e: (0, 0)),
        ],
        out_specs=pl.BlockSpec((T, D), lambda e: (0, 0)),
        out_shape=jax.ShapeDtypeStruct((T, D), jnp.float32),
    )(ctx, W1, b1.reshape(E, 1, DFF), W2, b2.reshape(E, 1, D), gates)

    out = pl.pallas_call(
        _stage_core,
        out_shape=jax.ShapeDtypeStruct((T, 1), jnp.float32),
    )(moe, W_core, b_core.reshape(1, D), W_h1, b_h1.reshape(1, 256),
      W_h2, b_h2.reshape(1, 1))

    return out
